# baseline (speedup 1.0000x reference)
import functools

import jax
import jax.numpy as jnp
from jax import lax
from jax.experimental import pallas as pl
from jax.experimental.pallas import tpu as pltpu
from jax.experimental.pallas import tpu_sc as plsc

_EMB = 32
_K = 8192
_CHUNK = 2048
_NCHUNK = _K // _CHUNK
_ROWS = 256
_N = 8192
_NT = _N // _ROWS
_VQ_COMMIT = 0.25

_NW = 32
_BPW = _N // _NW
_ISEG = 128


def _argmin_body(x_ref, w_ref, rs_ref, ind_ref, diff_ref, acc_ref):
    i = pl.program_id(0)
    x = x_ref[...]
    w = w_ref[...]
    rs = rs_ref[...]

    cs = jnp.sum(w ** 2, axis=0, keepdims=True)
    mm = jnp.dot(x, w, preferred_element_type=jnp.float32)

    lane_if = jax.lax.broadcasted_iota(
        jnp.int32, (_ROWS, 128), 1).astype(jnp.float32)
    nblk = _CHUNK // 128
    mcs = []
    afs = []
    for c in range(_NCHUNK):
        base = c * _CHUNK
        bestv = rs - 2.0 * mm[:, base:base + 128] + cs[:, base:base + 128]
        besti = lane_if
        for b in range(1, nblk):
            sl = slice(base + b * 128, base + (b + 1) * 128)
            blk = rs - 2.0 * mm[:, sl] + cs[:, sl]
            lt = blk < bestv
            bestv = jnp.where(lt, blk, bestv)
            besti = jnp.where(lt, lane_if + (b * 128.0), besti)
        mv = jnp.min(bestv, axis=1, keepdims=True)
        acf = jnp.min(jnp.where(bestv == mv, besti, float(_CHUNK)), axis=1)
        mcs.append(mv[:, 0])
        afs.append(acf + float(base))

    def _bf(v):
        return v.astype(jnp.bfloat16).astype(jnp.float32)

    t1 = mcs[1] < _bf(mcs[0])
    le = jnp.where(t1, mcs[1], mcs[0])
    li = jnp.where(t1, afs[1], afs[0])
    t3 = mcs[3] < _bf(mcs[2])
    re = jnp.where(t3, mcs[3], mcs[2])
    ri = jnp.where(t3, afs[3], afs[2])
    tr = re < _bf(le)
    selv = jnp.where(tr, re, le)
    accif = jnp.where(tr, ri, li)

    ind_ref[0, 0, :] = accif.astype(jnp.int32)
    psq = jnp.sum(selv)

    @pl.when(i == 0)
    def _():
        acc_ref[0, 0] = psq

    @pl.when(i > 0)
    def _():
        acc_ref[0, 0] += psq

    @pl.when(i == _NT - 1)
    def _():
        total = acc_ref[0, 0] / (_N * _EMB)
        diff_ref[...] = jnp.full((1, 1), _VQ_COMMIT * total + total,
                                 jnp.float32)


_sc_mesh = plsc.VectorSubcoreMesh(core_axis_name="c", subcore_axis_name="s")


_ROW128 = 128


@functools.partial(
    pl.kernel, mesh=_sc_mesh,
    out_type=[jax.ShapeDtypeStruct((_N, _ROW128), jnp.float32),
              jax.ShapeDtypeStruct((2, _K), jnp.float32)],
    scratch_types=[pltpu.VMEM((_BPW,), jnp.int32),
                   pltpu.VMEM((_BPW, _ROW128), jnp.float32),
                   pltpu.VMEM((_BPW,), jnp.float32),
                   pltpu.VMEM_SHARED((_K,), jnp.float32),
                   pltpu.SemaphoreType.DMA])
def _sc_gather_hist(table_hbm, idx_hbm, zeros_hbm, ones_hbm, q_hbm, cnt_hbm,
                    idx_v, rows_v, ones_v, shared, sem):
    c = lax.axis_index("c")
    s = lax.axis_index("s")
    wid = s * 2 + c
    base = wid * _BPW
    pltpu.sync_copy(idx_hbm.at[pl.ds(base, _BPW)], idx_v)
    pltpu.sync_copy(ones_hbm.at[pl.ds(0, _BPW)], ones_v)
    for j in range(_BPW // _ISEG):
        pltpu.async_copy(
            table_hbm.at[idx_v.at[pl.ds(j * _ISEG, _ISEG)]],
            rows_v.at[pl.ds(j * _ISEG, _ISEG)], sem).wait()
    pltpu.sync_copy(rows_v, q_hbm.at[pl.ds(base, _BPW)])

    @pl.when(s == 0)
    def _():
        pltpu.sync_copy(zeros_hbm, shared)

    plsc.subcore_barrier()
    for j in range(_BPW // _ISEG):
        pltpu.sync_copy(ones_v.at[pl.ds(j * _ISEG, _ISEG)],
                        shared.at[idx_v.at[pl.ds(j * _ISEG, _ISEG)]],
                        add=True)
    plsc.subcore_barrier()

    @pl.when(s == 0)
    def _():
        pltpu.sync_copy(shared, cnt_hbm.at[c])


def _finish_body(q_ref, x_ref, cnt_ref, quant_ref, perp_ref):
    q = q_ref[:, :_EMB]
    x = x_ref[...]
    quant_ref[...] = x + (q - x)
    counts = cnt_ref[0:1, :] + cnt_ref[1:2, :]
    avg = counts / _N
    ent = jnp.sum(avg * jnp.log(avg + 1e-10))
    perp_ref[...] = jnp.exp(-jnp.full((1, 1), ent, jnp.float32))


def kernel(input, W):
    x = jnp.swapaxes(input, 1, -1)
    flat = x.reshape(-1, _EMB)
    x2 = flat ** 2
    t = x2
    for _lvl in range(5):
        t = t[:, ::2] + t[:, 1::2]
    rowsq = t
    table = jnp.pad(jnp.swapaxes(W, 0, 1),
                    ((0, 0), (0, _ROW128 - _EMB)))

    ind3, diff = pl.pallas_call(
        _argmin_body,
        grid=(_NT,),
        in_specs=[
            pl.BlockSpec((_ROWS, _EMB), lambda i: (i, 0)),
            pl.BlockSpec((_EMB, _K), lambda i: (0, 0)),
            pl.BlockSpec((_ROWS, 1), lambda i: (i, 0)),
        ],
        out_specs=[
            pl.BlockSpec((1, 1, _ROWS), lambda i: (i, 0, 0)),
            pl.BlockSpec((1, 1), lambda i: (0, 0)),
        ],
        out_shape=[
            jax.ShapeDtypeStruct((_NT, 1, _ROWS), jnp.int32),
            jax.ShapeDtypeStruct((1, 1), jnp.float32),
        ],
        scratch_shapes=[
            pltpu.SMEM((1, 1), jnp.float32),
        ],
    )(flat, W, rowsq)

    ind_flat = ind3.reshape(_N)
    zeros = jnp.zeros((_K,), jnp.float32)
    ones = jnp.ones((_BPW,), jnp.float32)
    q_raw, cnt2 = _sc_gather_hist(table, ind_flat, zeros, ones)

    quant_flat, perp = pl.pallas_call(
        _finish_body,
        out_shape=[
            jax.ShapeDtypeStruct((_N, _EMB), jnp.float32),
            jax.ShapeDtypeStruct((1, 1), jnp.float32),
        ],
    )(q_raw, flat, cnt2)

    quantize = jnp.swapaxes(quant_flat.reshape(x.shape), 1, -1)
    ind_r = ind_flat.reshape(x.shape[:-1])
    return (quantize, diff.reshape(()), ind_r, perp.reshape(()))


# speedup vs baseline: 1.0160x; 1.0160x over previous
import functools

import jax
import jax.numpy as jnp
from jax import lax
from jax.experimental import pallas as pl
from jax.experimental.pallas import tpu as pltpu
from jax.experimental.pallas import tpu_sc as plsc

_EMB = 32
_K = 8192
_CHUNK = 2048
_NCHUNK = _K // _CHUNK
_ROWS = 256
_N = 8192
_NT = _N // _ROWS
_VQ_COMMIT = 0.25

_NW = 32
_BPW = _N // _NW
_ISEG = 128


def _argmin_body(x_ref, w_ref, rs_ref, ind_ref, diff_ref, acc_ref):
    i = pl.program_id(0)
    x = x_ref[...]
    w = w_ref[...]
    rs = rs_ref[...]

    cs = jnp.sum(w ** 2, axis=0, keepdims=True)
    mm = jnp.dot(x, w, preferred_element_type=jnp.float32)

    lane_if = jax.lax.broadcasted_iota(
        jnp.int32, (_ROWS, 128), 1).astype(jnp.float32)
    nblk = _CHUNK // 128
    mcs = []
    afs = []
    for c in range(_NCHUNK):
        base = c * _CHUNK
        bestv = rs - 2.0 * mm[:, base:base + 128] + cs[:, base:base + 128]
        besti = lane_if
        for b in range(1, nblk):
            sl = slice(base + b * 128, base + (b + 1) * 128)
            blk = rs - 2.0 * mm[:, sl] + cs[:, sl]
            lt = blk < bestv
            bestv = jnp.where(lt, blk, bestv)
            besti = jnp.where(lt, lane_if + (b * 128.0), besti)
        mv = jnp.min(bestv, axis=1, keepdims=True)
        acf = jnp.min(jnp.where(bestv == mv, besti, float(_CHUNK)), axis=1)
        mcs.append(mv[:, 0])
        afs.append(acf + float(base))

    def _bf(v):
        return v.astype(jnp.bfloat16).astype(jnp.float32)

    t1 = mcs[1] < _bf(mcs[0])
    le = jnp.where(t1, mcs[1], mcs[0])
    li = jnp.where(t1, afs[1], afs[0])
    t3 = mcs[3] < _bf(mcs[2])
    re = jnp.where(t3, mcs[3], mcs[2])
    ri = jnp.where(t3, afs[3], afs[2])
    tr = re < _bf(le)
    selv = jnp.where(tr, re, le)
    accif = jnp.where(tr, ri, li)

    ind_ref[0, 0, :] = accif.astype(jnp.int32)
    psq = jnp.sum(selv)

    @pl.when(i == 0)
    def _():
        acc_ref[0, 0] = psq

    @pl.when(i > 0)
    def _():
        acc_ref[0, 0] += psq

    @pl.when(i == _NT - 1)
    def _():
        total = acc_ref[0, 0] / (_N * _EMB)
        diff_ref[...] = jnp.full((1, 1), _VQ_COMMIT * total + total,
                                 jnp.float32)


_sc_mesh = plsc.VectorSubcoreMesh(core_axis_name="c", subcore_axis_name="s")


_ROW128 = 128


@functools.partial(
    pl.kernel, mesh=_sc_mesh,
    out_type=[jax.ShapeDtypeStruct((_N, _ROW128), jnp.float32),
              jax.ShapeDtypeStruct((2, _K), jnp.float32)],
    scratch_types=[pltpu.VMEM((_BPW,), jnp.int32),
                   pltpu.VMEM((_BPW, _ROW128), jnp.float32),
                   pltpu.VMEM((_BPW,), jnp.float32),
                   pltpu.VMEM_SHARED((_K,), jnp.float32),
                   pltpu.SemaphoreType.DMA])
def _sc_gather_hist(table_hbm, idx_hbm, zeros_hbm, ones_hbm, q_hbm, cnt_hbm,
                    idx_v, rows_v, ones_v, shared, sem):
    c = lax.axis_index("c")
    s = lax.axis_index("s")
    wid = s * 2 + c
    base = wid * _BPW
    pltpu.sync_copy(idx_hbm.at[pl.ds(base, _BPW)], idx_v)
    pltpu.sync_copy(ones_hbm.at[pl.ds(0, _BPW)], ones_v)
    for j in range(_BPW // _ISEG):
        pltpu.async_copy(
            table_hbm.at[idx_v.at[pl.ds(j * _ISEG, _ISEG)]],
            rows_v.at[pl.ds(j * _ISEG, _ISEG)], sem).wait()
    pltpu.sync_copy(rows_v, q_hbm.at[pl.ds(base, _BPW)])

    @pl.when(s == 0)
    def _():
        pltpu.sync_copy(zeros_hbm, shared)

    plsc.subcore_barrier()
    for j in range(_BPW // _ISEG):
        pltpu.sync_copy(ones_v.at[pl.ds(j * _ISEG, _ISEG)],
                        shared.at[idx_v.at[pl.ds(j * _ISEG, _ISEG)]],
                        add=True)
    plsc.subcore_barrier()

    @pl.when(s == 0)
    def _():
        pltpu.sync_copy(shared, cnt_hbm.at[c])


def _finish_body(cnt_ref, perp_ref):
    counts = cnt_ref[0:1, :] + cnt_ref[1:2, :]
    avg = counts / _N
    ent = jnp.sum(avg * jnp.log(avg + 1e-10))
    perp_ref[...] = jnp.exp(-jnp.full((1, 1), ent, jnp.float32))


def kernel(input, W):
    x = jnp.swapaxes(input, 1, -1)
    flat = x.reshape(-1, _EMB)
    x2 = flat ** 2
    t = x2
    for _lvl in range(5):
        t = t[:, ::2] + t[:, 1::2]
    rowsq = t
    table = jnp.pad(jnp.swapaxes(W, 0, 1),
                    ((0, 0), (0, _ROW128 - _EMB)))

    ind3, diff = pl.pallas_call(
        _argmin_body,
        grid=(_NT,),
        in_specs=[
            pl.BlockSpec((_ROWS, _EMB), lambda i: (i, 0)),
            pl.BlockSpec((_EMB, _K), lambda i: (0, 0)),
            pl.BlockSpec((_ROWS, 1), lambda i: (i, 0)),
        ],
        out_specs=[
            pl.BlockSpec((1, 1, _ROWS), lambda i: (i, 0, 0)),
            pl.BlockSpec((1, 1), lambda i: (0, 0)),
        ],
        out_shape=[
            jax.ShapeDtypeStruct((_NT, 1, _ROWS), jnp.int32),
            jax.ShapeDtypeStruct((1, 1), jnp.float32),
        ],
        scratch_shapes=[
            pltpu.SMEM((1, 1), jnp.float32),
        ],
    )(flat, W, rowsq)

    ind_flat = ind3.reshape(_N)
    zeros = jnp.zeros((_K,), jnp.float32)
    ones = jnp.ones((_BPW,), jnp.float32)
    q_raw, cnt2 = _sc_gather_hist(table, ind_flat, zeros, ones)

    perp = pl.pallas_call(
        _finish_body,
        out_shape=jax.ShapeDtypeStruct((1, 1), jnp.float32),
    )(cnt2)

    quantize = jnp.swapaxes(q_raw[:, :_EMB].reshape(x.shape), 1, -1)
    ind_r = ind_flat.reshape(x.shape[:-1])
    return (quantize, diff.reshape(()), ind_r, perp.reshape(()))


# speedup vs baseline: 1.0164x; 1.0004x over previous
import functools

import jax
import jax.numpy as jnp
from jax import lax
from jax.experimental import pallas as pl
from jax.experimental.pallas import tpu as pltpu
from jax.experimental.pallas import tpu_sc as plsc

_EMB = 32
_K = 8192
_CHUNK = 2048
_NCHUNK = _K // _CHUNK
_ROWS = 256
_N = 8192
_NT = _N // _ROWS
_VQ_COMMIT = 0.25

_NW = 32
_BPW = _N // _NW
_ISEG = 128


def _argmin_body(x_ref, w_ref, rs_ref, ind_ref, psq_ref):
    x = x_ref[...]
    w = w_ref[...]
    rs = rs_ref[...]

    cs = jnp.sum(w ** 2, axis=0, keepdims=True)
    mm = jnp.dot(x, w, preferred_element_type=jnp.float32)

    lane_if = jax.lax.broadcasted_iota(
        jnp.int32, (_ROWS, 128), 1).astype(jnp.float32)
    nblk = _CHUNK // 128
    mcs = []
    afs = []
    for c in range(_NCHUNK):
        base = c * _CHUNK
        bestv = rs - 2.0 * mm[:, base:base + 128] + cs[:, base:base + 128]
        besti = lane_if
        for b in range(1, nblk):
            sl = slice(base + b * 128, base + (b + 1) * 128)
            blk = rs - 2.0 * mm[:, sl] + cs[:, sl]
            lt = blk < bestv
            bestv = jnp.where(lt, blk, bestv)
            besti = jnp.where(lt, lane_if + (b * 128.0), besti)
        mv = jnp.min(bestv, axis=1, keepdims=True)
        acf = jnp.min(jnp.where(bestv == mv, besti, float(_CHUNK)), axis=1)
        mcs.append(mv[:, 0])
        afs.append(acf + float(base))

    def _bf(v):
        return v.astype(jnp.bfloat16).astype(jnp.float32)

    t1 = mcs[1] < _bf(mcs[0])
    le = jnp.where(t1, mcs[1], mcs[0])
    li = jnp.where(t1, afs[1], afs[0])
    t3 = mcs[3] < _bf(mcs[2])
    re = jnp.where(t3, mcs[3], mcs[2])
    ri = jnp.where(t3, afs[3], afs[2])
    tr = re < _bf(le)
    selv = jnp.where(tr, re, le)
    accif = jnp.where(tr, ri, li)

    ind_ref[0, 0, :] = accif.astype(jnp.int32)
    psq_ref[...] = jnp.full((1, 1, 1), jnp.sum(selv), jnp.float32)


_sc_mesh = plsc.VectorSubcoreMesh(core_axis_name="c", subcore_axis_name="s")


_ROW128 = 128


@functools.partial(
    pl.kernel, mesh=_sc_mesh,
    out_type=[jax.ShapeDtypeStruct((_N, _ROW128), jnp.float32),
              jax.ShapeDtypeStruct((2, _K), jnp.float32)],
    scratch_types=[pltpu.VMEM((_BPW,), jnp.int32),
                   pltpu.VMEM((_BPW, _ROW128), jnp.float32),
                   pltpu.VMEM((_BPW,), jnp.float32),
                   pltpu.VMEM_SHARED((_K,), jnp.float32),
                   pltpu.SemaphoreType.DMA])
def _sc_gather_hist(table_hbm, idx_hbm, zeros_hbm, ones_hbm, q_hbm, cnt_hbm,
                    idx_v, rows_v, ones_v, shared, sem):
    c = lax.axis_index("c")
    s = lax.axis_index("s")
    wid = s * 2 + c
    base = wid * _BPW
    pltpu.sync_copy(idx_hbm.at[pl.ds(base, _BPW)], idx_v)
    pltpu.sync_copy(ones_hbm.at[pl.ds(0, _BPW)], ones_v)
    for j in range(_BPW // _ISEG):
        pltpu.async_copy(
            table_hbm.at[idx_v.at[pl.ds(j * _ISEG, _ISEG)]],
            rows_v.at[pl.ds(j * _ISEG, _ISEG)], sem).wait()
    pltpu.sync_copy(rows_v, q_hbm.at[pl.ds(base, _BPW)])

    @pl.when(s == 0)
    def _():
        pltpu.sync_copy(zeros_hbm, shared)

    plsc.subcore_barrier()
    for j in range(_BPW // _ISEG):
        pltpu.sync_copy(ones_v.at[pl.ds(j * _ISEG, _ISEG)],
                        shared.at[idx_v.at[pl.ds(j * _ISEG, _ISEG)]],
                        add=True)
    plsc.subcore_barrier()

    @pl.when(s == 0)
    def _():
        pltpu.sync_copy(shared, cnt_hbm.at[c])


def _finish_body(cnt_ref, psq_ref, perp_ref, diff_ref):
    counts = cnt_ref[0:1, :] + cnt_ref[1:2, :]
    avg = counts / _N
    ent = jnp.sum(avg * jnp.log(avg + 1e-10))
    perp_ref[...] = jnp.exp(-jnp.full((1, 1), ent, jnp.float32))
    total = jnp.sum(psq_ref[...]) / (_N * _EMB)
    diff_ref[...] = jnp.full((1, 1), _VQ_COMMIT * total + total, jnp.float32)


def kernel(input, W):
    x = jnp.swapaxes(input, 1, -1)
    flat = x.reshape(-1, _EMB)
    x2 = flat ** 2
    t = x2
    for _lvl in range(5):
        t = t[:, ::2] + t[:, 1::2]
    rowsq = t
    table = jnp.pad(jnp.swapaxes(W, 0, 1),
                    ((0, 0), (0, _ROW128 - _EMB)))

    ind3, psq = pl.pallas_call(
        _argmin_body,
        grid=(_NT,),
        in_specs=[
            pl.BlockSpec((_ROWS, _EMB), lambda i: (i, 0)),
            pl.BlockSpec((_EMB, _K), lambda i: (0, 0)),
            pl.BlockSpec((_ROWS, 1), lambda i: (i, 0)),
        ],
        out_specs=[
            pl.BlockSpec((1, 1, _ROWS), lambda i: (i, 0, 0)),
            pl.BlockSpec((1, 1, 1), lambda i: (i, 0, 0)),
        ],
        out_shape=[
            jax.ShapeDtypeStruct((_NT, 1, _ROWS), jnp.int32),
            jax.ShapeDtypeStruct((_NT, 1, 1), jnp.float32),
        ],
        compiler_params=pltpu.CompilerParams(
            dimension_semantics=("parallel",)),
    )(flat, W, rowsq)

    ind_flat = ind3.reshape(_N)
    zeros = jnp.zeros((_K,), jnp.float32)
    ones = jnp.ones((_BPW,), jnp.float32)
    q_raw, cnt2 = _sc_gather_hist(table, ind_flat, zeros, ones)

    perp, diff = pl.pallas_call(
        _finish_body,
        out_shape=[
            jax.ShapeDtypeStruct((1, 1), jnp.float32),
            jax.ShapeDtypeStruct((1, 1), jnp.float32),
        ],
    )(cnt2, psq.reshape(_NT, 1))

    quantize = jnp.swapaxes(q_raw[:, :_EMB].reshape(x.shape), 1, -1)
    ind_r = ind_flat.reshape(x.shape[:-1])
    return (quantize, diff.reshape(()), ind_r, perp.reshape(()))


# speedup vs baseline: 1.0391x; 1.0223x over previous
import functools

import jax
import jax.numpy as jnp
from jax import lax
from jax.experimental import pallas as pl
from jax.experimental.pallas import tpu as pltpu
from jax.experimental.pallas import tpu_sc as plsc

_EMB = 32
_K = 8192
_CHUNK = 2048
_NCHUNK = _K // _CHUNK
_ROWS = 512
_N = 8192
_NT = _N // _ROWS
_VQ_COMMIT = 0.25

_NW = 32
_BPW = _N // _NW
_ISEG = 128


def _argmin_body(x_ref, w_ref, rs_ref, ind_ref, psq_ref):
    x = x_ref[...]
    w = w_ref[...]
    rs = rs_ref[...]

    cs = jnp.sum(w ** 2, axis=0, keepdims=True)
    mm = jnp.dot(x, w, preferred_element_type=jnp.float32)

    lane_if = jax.lax.broadcasted_iota(
        jnp.int32, (_ROWS, 128), 1).astype(jnp.float32)
    nblk = _CHUNK // 128
    mcs = []
    afs = []
    for c in range(_NCHUNK):
        base = c * _CHUNK
        bestv = rs - 2.0 * mm[:, base:base + 128] + cs[:, base:base + 128]
        besti = lane_if
        for b in range(1, nblk):
            sl = slice(base + b * 128, base + (b + 1) * 128)
            blk = rs - 2.0 * mm[:, sl] + cs[:, sl]
            lt = blk < bestv
            bestv = jnp.where(lt, blk, bestv)
            besti = jnp.where(lt, lane_if + (b * 128.0), besti)
        mv = jnp.min(bestv, axis=1, keepdims=True)
        acf = jnp.min(jnp.where(bestv == mv, besti, float(_CHUNK)), axis=1)
        mcs.append(mv[:, 0])
        afs.append(acf + float(base))

    def _bf(v):
        return v.astype(jnp.bfloat16).astype(jnp.float32)

    t1 = mcs[1] < _bf(mcs[0])
    le = jnp.where(t1, mcs[1], mcs[0])
    li = jnp.where(t1, afs[1], afs[0])
    t3 = mcs[3] < _bf(mcs[2])
    re = jnp.where(t3, mcs[3], mcs[2])
    ri = jnp.where(t3, afs[3], afs[2])
    tr = re < _bf(le)
    selv = jnp.where(tr, re, le)
    accif = jnp.where(tr, ri, li)

    ind_ref[0, 0, :] = accif.astype(jnp.int32)
    psq_ref[...] = jnp.full((1, 1, 1), jnp.sum(selv), jnp.float32)


_sc_mesh = plsc.VectorSubcoreMesh(core_axis_name="c", subcore_axis_name="s")


_ROW128 = 128


@functools.partial(
    pl.kernel, mesh=_sc_mesh,
    out_type=[jax.ShapeDtypeStruct((_N, _ROW128), jnp.float32),
              jax.ShapeDtypeStruct((2, _K), jnp.float32)],
    scratch_types=[pltpu.VMEM((_BPW,), jnp.int32),
                   pltpu.VMEM((_BPW, _ROW128), jnp.float32),
                   pltpu.VMEM((_BPW,), jnp.float32),
                   pltpu.VMEM_SHARED((_K,), jnp.float32),
                   pltpu.SemaphoreType.DMA])
def _sc_gather_hist(table_hbm, idx_hbm, zeros_hbm, ones_hbm, q_hbm, cnt_hbm,
                    idx_v, rows_v, ones_v, shared, sem):
    c = lax.axis_index("c")
    s = lax.axis_index("s")
    wid = s * 2 + c
    base = wid * _BPW
    pltpu.sync_copy(idx_hbm.at[pl.ds(base, _BPW)], idx_v)
    pltpu.sync_copy(ones_hbm.at[pl.ds(0, _BPW)], ones_v)
    for j in range(_BPW // _ISEG):
        pltpu.async_copy(
            table_hbm.at[idx_v.at[pl.ds(j * _ISEG, _ISEG)]],
            rows_v.at[pl.ds(j * _ISEG, _ISEG)], sem).wait()
    pltpu.sync_copy(rows_v, q_hbm.at[pl.ds(base, _BPW)])

    @pl.when(s == 0)
    def _():
        pltpu.sync_copy(zeros_hbm, shared)

    plsc.subcore_barrier()
    for j in range(_BPW // _ISEG):
        pltpu.sync_copy(ones_v.at[pl.ds(j * _ISEG, _ISEG)],
                        shared.at[idx_v.at[pl.ds(j * _ISEG, _ISEG)]],
                        add=True)
    plsc.subcore_barrier()

    @pl.when(s == 0)
    def _():
        pltpu.sync_copy(shared, cnt_hbm.at[c])


def _finish_body(cnt_ref, psq_ref, perp_ref, diff_ref):
    counts = cnt_ref[0:1, :] + cnt_ref[1:2, :]
    avg = counts / _N
    ent = jnp.sum(avg * jnp.log(avg + 1e-10))
    perp_ref[...] = jnp.exp(-jnp.full((1, 1), ent, jnp.float32))
    total = jnp.sum(psq_ref[...]) / (_N * _EMB)
    diff_ref[...] = jnp.full((1, 1), _VQ_COMMIT * total + total, jnp.float32)


def kernel(input, W):
    x = jnp.swapaxes(input, 1, -1)
    flat = x.reshape(-1, _EMB)
    x2 = flat ** 2
    t = x2
    for _lvl in range(5):
        t = t[:, ::2] + t[:, 1::2]
    rowsq = t
    table = jnp.pad(jnp.swapaxes(W, 0, 1),
                    ((0, 0), (0, _ROW128 - _EMB)))

    ind3, psq = pl.pallas_call(
        _argmin_body,
        grid=(_NT,),
        in_specs=[
            pl.BlockSpec((_ROWS, _EMB), lambda i: (i, 0)),
            pl.BlockSpec((_EMB, _K), lambda i: (0, 0)),
            pl.BlockSpec((_ROWS, 1), lambda i: (i, 0)),
        ],
        out_specs=[
            pl.BlockSpec((1, 1, _ROWS), lambda i: (i, 0, 0)),
            pl.BlockSpec((1, 1, 1), lambda i: (i, 0, 0)),
        ],
        out_shape=[
            jax.ShapeDtypeStruct((_NT, 1, _ROWS), jnp.int32),
            jax.ShapeDtypeStruct((_NT, 1, 1), jnp.float32),
        ],
        compiler_params=pltpu.CompilerParams(
            dimension_semantics=("parallel",)),
    )(flat, W, rowsq)

    ind_flat = ind3.reshape(_N)
    zeros = jnp.zeros((_K,), jnp.float32)
    ones = jnp.ones((_BPW,), jnp.float32)
    q_raw, cnt2 = _sc_gather_hist(table, ind_flat, zeros, ones)

    perp, diff = pl.pallas_call(
        _finish_body,
        out_shape=[
            jax.ShapeDtypeStruct((1, 1), jnp.float32),
            jax.ShapeDtypeStruct((1, 1), jnp.float32),
        ],
    )(cnt2, psq.reshape(_NT, 1))

    quantize = jnp.swapaxes(q_raw[:, :_EMB].reshape(x.shape), 1, -1)
    ind_r = ind_flat.reshape(x.shape[:-1])
    return (quantize, diff.reshape(()), ind_r, perp.reshape(()))
